# baseline (speedup 1.0000x reference)
import functools

import jax
import jax.numpy as jnp
from jax import lax
from jax.experimental import pallas as pl
from jax.experimental.pallas import tpu as pltpu
from jax.experimental.pallas import tpu_sc as plsc

_B = 64
_NCHUNKS = 768
_CHUNKS_PER_B = _NCHUNKS // _B
_CROWS = 64
_CCOLS = 256
_NW = 32
_PER_W = _NCHUNKS // _NW
_BATCHES_PER_W = _PER_W // _CHUNKS_PER_B


def _sc_body(t_hbm, sa_hbm, som_hbm, x_hbm, n_hbm, o_hbm,
             ib16, tb16, sa16, som16,
             xb0, xb1, nb0, nb1, ob0, ob1,
             gsem, xsem0, xsem1, nsem0, nsem1, osem0, osem1):
    wid = lax.axis_index("s") * 2 + lax.axis_index("c")
    xb = (xb0, xb1)
    nb = (nb0, nb1)
    ob = (ob0, ob1)
    xsem = (xsem0, xsem1)
    nsem = (nsem0, nsem1)
    osem = (osem0, osem1)

    b0 = wid * _BATCHES_PER_W
    for k in range(_BATCHES_PER_W):
        ib16[k] = jnp.full((16,), b0 + k, dtype=jnp.int32)
    for k in range(_BATCHES_PER_W):
        pltpu.async_copy(t_hbm.at[ib16.at[k]], tb16.at[k], gsem).wait()
    for k in range(_BATCHES_PER_W):
        pltpu.async_copy(sa_hbm.at[tb16.at[k]], sa16.at[k], gsem).wait()
        pltpu.async_copy(som_hbm.at[tb16.at[k]], som16.at[k], gsem).wait()

    def chunk_idx(j):
        return wid * _PER_W + j

    def in_copies(j, s):
        m = chunk_idx(j)
        cx = pltpu.make_async_copy(x_hbm.at[m], xb[s], xsem[s])
        cn = pltpu.make_async_copy(n_hbm.at[m], nb[s], nsem[s])
        return cx, cn

    def out_copy(j, s):
        return pltpu.make_async_copy(ob[s], o_hbm.at[chunk_idx(j)], osem[s])

    for j in range(2):
        cx, cn = in_copies(j, j)
        cx.start()
        cn.start()

    for j in range(_PER_W):
        s = j % 2
        cx, cn = in_copies(j, s)
        cx.wait()
        cn.wait()
        if j >= 2:
            out_copy(j - 2, s).wait()
        k = j // _CHUNKS_PER_B
        sa_vec = sa16[k]
        som_vec = som16[k]

        def row_body(r, carry):
            for g in range(_CCOLS // 16):
                cs = pl.ds(g * 16, 16)
                ob[s][r, cs] = sa_vec * xb[s][r, cs] + som_vec * nb[s][r, cs]
            return carry

        lax.fori_loop(0, _CROWS, row_body, 0)
        out_copy(j, s).start()
        nxt = j + 2
        if nxt < _PER_W:
            cx2, cn2 = in_copies(nxt, s)
            cx2.start()
            cn2.start()

    for j in range(_PER_W - 2, _PER_W):
        out_copy(j, j % 2).wait()


@functools.partial(
    pl.kernel,
    mesh=plsc.VectorSubcoreMesh(core_axis_name="c", subcore_axis_name="s"),
    out_type=jax.ShapeDtypeStruct((_NCHUNKS, _CROWS, _CCOLS), jnp.float32),
    scratch_types=[
        pltpu.VMEM((_BATCHES_PER_W, 16), jnp.int32),
        pltpu.VMEM((_BATCHES_PER_W, 16), jnp.int32),
        pltpu.VMEM((_BATCHES_PER_W, 16), jnp.float32),
        pltpu.VMEM((_BATCHES_PER_W, 16), jnp.float32),
        pltpu.VMEM((_CROWS, _CCOLS), jnp.float32),
        pltpu.VMEM((_CROWS, _CCOLS), jnp.float32),
        pltpu.VMEM((_CROWS, _CCOLS), jnp.float32),
        pltpu.VMEM((_CROWS, _CCOLS), jnp.float32),
        pltpu.VMEM((_CROWS, _CCOLS), jnp.float32),
        pltpu.VMEM((_CROWS, _CCOLS), jnp.float32),
        pltpu.SemaphoreType.DMA,
        pltpu.SemaphoreType.DMA,
        pltpu.SemaphoreType.DMA,
        pltpu.SemaphoreType.DMA,
        pltpu.SemaphoreType.DMA,
        pltpu.SemaphoreType.DMA,
        pltpu.SemaphoreType.DMA,
    ],
)
def _sc_qsample(t_hbm, sa_hbm, som_hbm, x_hbm, n_hbm, o_hbm, *scratch):
    _sc_body(t_hbm, sa_hbm, som_hbm, x_hbm, n_hbm, o_hbm, *scratch)


def kernel(x_start, t, noise, sqrt_alphas_cumprod, sqrt_one_minus_alphas_cumprod):
    b, ch, h, w = x_start.shape
    x3 = x_start.reshape(_NCHUNKS, _CROWS, _CCOLS)
    n3 = noise.reshape(_NCHUNKS, _CROWS, _CCOLS)
    out = _sc_qsample(t.astype(jnp.int32), sqrt_alphas_cumprod,
                      sqrt_one_minus_alphas_cumprod, x3, n3)
    return out.reshape(b, ch, h, w)


# speedup vs baseline: 1.2723x; 1.2723x over previous
import functools

import jax
import jax.numpy as jnp
from jax import lax
from jax.experimental import pallas as pl
from jax.experimental.pallas import tpu as pltpu
from jax.experimental.pallas import tpu_sc as plsc

_B = 64
_NBUF = 6


def _sc_gather_body(t_hbm, sa_hbm, som_hbm, sag_hbm, somg_hbm,
                    tv, sag, somg, sem):
    wid = lax.axis_index("s") * 2 + lax.axis_index("c")

    @pl.when(wid == 0)
    def _():
        pltpu.make_async_copy(t_hbm, tv, sem).start()
        pltpu.make_async_copy(t_hbm, tv, sem).wait()
        pltpu.async_copy(sa_hbm.at[tv], sag, sem).wait()
        pltpu.async_copy(som_hbm.at[tv], somg, sem).wait()
        pltpu.make_async_copy(sag, sag_hbm, sem).start()
        pltpu.make_async_copy(somg, somg_hbm, sem).start()
        pltpu.make_async_copy(sag, sag_hbm, sem).wait()
        pltpu.make_async_copy(somg, somg_hbm, sem).wait()


@functools.partial(
    pl.kernel,
    mesh=plsc.VectorSubcoreMesh(core_axis_name="c", subcore_axis_name="s"),
    out_type=(
        jax.ShapeDtypeStruct((_B,), jnp.float32),
        jax.ShapeDtypeStruct((_B,), jnp.float32),
    ),
    scratch_types=[
        pltpu.VMEM((_B,), jnp.int32),
        pltpu.VMEM((_B,), jnp.float32),
        pltpu.VMEM((_B,), jnp.float32),
        pltpu.SemaphoreType.DMA,
    ],
)
def _sc_gather(t_hbm, sa_hbm, som_hbm, sag_hbm, somg_hbm, *scratch):
    _sc_gather_body(t_hbm, sa_hbm, som_hbm, sag_hbm, somg_hbm, *scratch)


def _tc_body(sag_ref, somg_ref, x_hbm, n_hbm, o_hbm,
             xb, nb, ob, xsem, nsem, osem):
    nchunks = sag_ref.shape[0]

    def in_copies(c, slot):
        cx = pltpu.make_async_copy(x_hbm.at[c], xb.at[slot], xsem.at[slot])
        cn = pltpu.make_async_copy(n_hbm.at[c], nb.at[slot], nsem.at[slot])
        return cx, cn

    def out_copy(c, slot):
        return pltpu.make_async_copy(ob.at[slot], o_hbm.at[c], osem.at[slot])

    for b in range(_NBUF):
        cx, cn = in_copies(b, b)
        cx.start()
        cn.start()

    for c in range(nchunks):
        slot = c % _NBUF
        cx, cn = in_copies(c, slot)
        cx.wait()
        cn.wait()
        if c >= _NBUF:
            out_copy(c - _NBUF, slot).wait()
        ob[slot] = sag_ref[c] * xb[slot] + somg_ref[c] * nb[slot]
        out_copy(c, slot).start()
        nxt = c + _NBUF
        if nxt < nchunks:
            cx2, cn2 = in_copies(nxt, slot)
            cx2.start()
            cn2.start()

    for c in range(max(nchunks - _NBUF, 0), nchunks):
        out_copy(c, c % _NBUF).wait()


def kernel(x_start, t, noise, sqrt_alphas_cumprod, sqrt_one_minus_alphas_cumprod):
    b, ch, h, w = x_start.shape
    sa_g, som_g = _sc_gather(t.astype(jnp.int32), sqrt_alphas_cumprod,
                             sqrt_one_minus_alphas_cumprod)
    smem = pl.BlockSpec(memory_space=pltpu.SMEM)
    hbm = pl.BlockSpec(memory_space=pltpu.MemorySpace.HBM)
    buf = pltpu.VMEM((_NBUF, ch, h, w), jnp.float32)
    return pl.pallas_call(
        _tc_body,
        in_specs=[smem, smem, hbm, hbm],
        out_specs=hbm,
        out_shape=jax.ShapeDtypeStruct((b, ch, h, w), jnp.float32),
        scratch_shapes=[
            buf, buf, buf,
            pltpu.SemaphoreType.DMA((_NBUF,)),
            pltpu.SemaphoreType.DMA((_NBUF,)),
            pltpu.SemaphoreType.DMA((_NBUF,)),
        ],
    )(sa_g, som_g, x_start, noise)


# speedup vs baseline: 1.8096x; 1.4223x over previous
import jax
import jax.numpy as jnp
from jax.experimental import pallas as pl
from jax.experimental.pallas import tpu as pltpu

_NSLAB = 768
_SROWS = 64
_SCOLS = 256
_SLABS_PER_CHUNK = 6
_NCHUNKS = _NSLAB // _SLABS_PER_CHUNK
_CHUNKS_PER_B = _NCHUNKS // 64
_NBUF = 12


def _qsample_body(t_ref, sa_ref, som_ref, x_hbm, n_hbm, o_hbm,
                  xb, nb, ob, xsem, nsem, osem):
    def in_copies(c, slot):
        sl = pl.ds(c * _SLABS_PER_CHUNK, _SLABS_PER_CHUNK)
        cx = pltpu.make_async_copy(x_hbm.at[sl], xb.at[slot], xsem.at[slot])
        cn = pltpu.make_async_copy(n_hbm.at[sl], nb.at[slot], nsem.at[slot])
        return cx, cn

    def out_copy(c, slot):
        sl = pl.ds(c * _SLABS_PER_CHUNK, _SLABS_PER_CHUNK)
        return pltpu.make_async_copy(ob.at[slot], o_hbm.at[sl], osem.at[slot])

    for b in range(_NBUF):
        cx, cn = in_copies(b, b)
        cx.start()
        cn.start()

    for c in range(_NCHUNKS):
        slot = c % _NBUF
        cx, cn = in_copies(c, slot)
        cx.wait()
        cn.wait()
        if c >= _NBUF:
            out_copy(c - _NBUF, slot).wait()
        tt = t_ref[c // _CHUNKS_PER_B]
        ob[slot] = sa_ref[tt] * xb[slot] + som_ref[tt] * nb[slot]
        out_copy(c, slot).start()
        nxt = c + _NBUF
        if nxt < _NCHUNKS:
            cx2, cn2 = in_copies(nxt, slot)
            cx2.start()
            cn2.start()

    for c in range(max(_NCHUNKS - _NBUF, 0), _NCHUNKS):
        out_copy(c, c % _NBUF).wait()


def kernel(x_start, t, noise, sqrt_alphas_cumprod, sqrt_one_minus_alphas_cumprod):
    shape4 = x_start.shape
    x3 = x_start.reshape(_NSLAB, _SROWS, _SCOLS)
    n3 = noise.reshape(_NSLAB, _SROWS, _SCOLS)
    smem = pl.BlockSpec(memory_space=pltpu.SMEM)
    hbm = pl.BlockSpec(memory_space=pltpu.MemorySpace.HBM)
    buf = pltpu.VMEM((_NBUF, _SLABS_PER_CHUNK, _SROWS, _SCOLS), jnp.float32)
    out = pl.pallas_call(
        _qsample_body,
        in_specs=[smem, smem, smem, hbm, hbm],
        out_specs=hbm,
        out_shape=jax.ShapeDtypeStruct((_NSLAB, _SROWS, _SCOLS), jnp.float32),
        scratch_shapes=[
            buf, buf, buf,
            pltpu.SemaphoreType.DMA((_NBUF,)),
            pltpu.SemaphoreType.DMA((_NBUF,)),
            pltpu.SemaphoreType.DMA((_NBUF,)),
        ],
    )(t.astype(jnp.int32), sqrt_alphas_cumprod, sqrt_one_minus_alphas_cumprod,
      x3, n3)
    return out.reshape(shape4)


# speedup vs baseline: 1.8113x; 1.0009x over previous
import jax
import jax.numpy as jnp
from jax.experimental import pallas as pl
from jax.experimental.pallas import tpu as pltpu

_NBUF = 6


def _qsample_body(t_ref, sa_ref, som_ref, x_hbm, n_hbm, o_hbm,
                  xb, nb, ob, xsem, nsem, osem):
    nchunks = t_ref.shape[0]

    def in_copies(c, slot):
        cx = pltpu.make_async_copy(x_hbm.at[c], xb.at[slot], xsem.at[slot])
        cn = pltpu.make_async_copy(n_hbm.at[c], nb.at[slot], nsem.at[slot])
        return cx, cn

    def out_copy(c, slot):
        return pltpu.make_async_copy(ob.at[slot], o_hbm.at[c], osem.at[slot])

    for b in range(_NBUF):
        cx, cn = in_copies(b, b)
        cx.start()
        cn.start()

    for c in range(nchunks):
        slot = c % _NBUF
        cx, cn = in_copies(c, slot)
        cx.wait()
        cn.wait()
        if c >= _NBUF:
            out_copy(c - _NBUF, slot).wait()
        tt = t_ref[c]
        ob[slot] = sa_ref[tt] * xb[slot] + som_ref[tt] * nb[slot]
        out_copy(c, slot).start()
        nxt = c + _NBUF
        if nxt < nchunks:
            cx2, cn2 = in_copies(nxt, slot)
            cx2.start()
            cn2.start()

    for c in range(max(nchunks - _NBUF, 0), nchunks):
        out_copy(c, c % _NBUF).wait()


def kernel(x_start, t, noise, sqrt_alphas_cumprod, sqrt_one_minus_alphas_cumprod):
    b, ch, h, w = x_start.shape
    smem = pl.BlockSpec(memory_space=pltpu.SMEM)
    hbm = pl.BlockSpec(memory_space=pltpu.MemorySpace.HBM)
    buf = pltpu.VMEM((_NBUF, ch, h, w), jnp.float32)
    return pl.pallas_call(
        _qsample_body,
        in_specs=[smem, smem, smem, hbm, hbm],
        out_specs=hbm,
        out_shape=jax.ShapeDtypeStruct((b, ch, h, w), jnp.float32),
        scratch_shapes=[
            buf, buf, buf,
            pltpu.SemaphoreType.DMA((_NBUF,)),
            pltpu.SemaphoreType.DMA((_NBUF,)),
            pltpu.SemaphoreType.DMA((_NBUF,)),
        ],
    )(t.astype(jnp.int32), sqrt_alphas_cumprod, sqrt_one_minus_alphas_cumprod,
      x_start, noise)
